# baseline (speedup 1.0000x reference)
import dataclasses

import numpy as np

import jax
import jax.numpy as jnp
from jax import lax
from jax.experimental import pallas as pl
from jax.experimental.pallas import tpu as pltpu
from jax.experimental.pallas import tpu_sc as plsc

N = 10000
E = 320000
NP = 10000
D = 128
DH = D // 2
NREL = 16
B = 4096

NC, NS = 2, 16
NW = NC * NS
EPS = E // NS
BLK = 80
NBLK = EPS // BLK
SP = 81920
TRASH = 80000

_PERM = np.empty((D,), np.int32)
for _h in range(2):
    for _g in range(2):
        _base = _h * 64 + _g * 32
        _PERM[_base + 2 * np.arange(16)] = _base + np.arange(16)
        _PERM[_base + 2 * np.arange(16) + 1] = _base + 16 + np.arange(16)


def _sc_compiler_params():
    cp = pltpu.CompilerParams()
    fields = pltpu.CompilerParams.__dataclass_fields__
    if "needs_layout_passes" in fields:
        cp = dataclasses.replace(cp, needs_layout_passes=False)
    if "use_tc_tiling_on_sc" in fields:
        cp = dataclasses.replace(cp, use_tc_tiling_on_sc=False)
    return cp


K1B = 1000


def _k1_body(id_ref, ent_ref, ir_ref, w1_ref, wr_ref, att_ref,
             gt_ref, at_ref, lt_ref, xph_ref, rp_ref, sn_ref, sr_ref):
    f32 = jnp.float32
    w1 = w1_ref[...]
    gp = jnp.dot(gt_ref[...], w1[32:64, :], preferred_element_type=f32)
    ap = jnp.dot(at_ref[...], w1[64:96, :], preferred_element_type=f32)
    lp = jnp.dot(lt_ref[...], w1[96:128, :], preferred_element_type=f32)
    ent = ent_ref[...]
    ohg = (ent[:, 0:1] == lax.broadcasted_iota(jnp.int32, (1, 3), 1)
           ).astype(f32)
    oha = (ent[:, 1:2] == lax.broadcasted_iota(jnp.int32, (1, 9), 1)
           ).astype(f32)
    ohl = (ent[:, 2:3] == lax.broadcasted_iota(jnp.int32, (1, 11), 1)
           ).astype(f32)
    xp = (jnp.dot(id_ref[...], w1[0:32, :], preferred_element_type=f32)
          + jnp.dot(ohg, gp, preferred_element_type=f32)
          + jnp.dot(oha, ap, preferred_element_type=f32)
          + jnp.dot(ohl, lp, preferred_element_type=f32))
    att = att_ref[...]
    xph_ref[:, 0, :] = xp[:, :DH].astype(jnp.bfloat16)
    xph_ref[:, 1, :] = xp[:, DH:].astype(jnp.bfloat16)
    sn_ref[...] = jnp.dot(xp, att, preferred_element_type=f32)

    @pl.when(pl.program_id(0) == 0)
    def _():
        rp = jnp.dot(ir_ref[...], wr_ref[...], preferred_element_type=f32)
        rp_ref[...] = rp
        sr_ref[...] = jnp.dot(rp, att, preferred_element_type=f32)


def _k1(id_embed, ent, init_rel, W1, Wr1, att2, gtab, atab, ltab):
    f32 = jnp.float32
    g = N // K1B
    return pl.pallas_call(
        _k1_body,
        grid=(g,),
        in_specs=[
            pl.BlockSpec((K1B, 32), lambda i: (i, 0)),
            pl.BlockSpec((K1B, 3), lambda i: (i, 0)),
            pl.BlockSpec((NREL, D), lambda i: (0, 0)),
            pl.BlockSpec((D, D), lambda i: (0, 0)),
            pl.BlockSpec((D, D), lambda i: (0, 0)),
            pl.BlockSpec((D, 1), lambda i: (0, 0)),
            pl.BlockSpec((3, 32), lambda i: (0, 0)),
            pl.BlockSpec((9, 32), lambda i: (0, 0)),
            pl.BlockSpec((11, 32), lambda i: (0, 0)),
        ],
        out_specs=[
            pl.BlockSpec((K1B, 2, DH), lambda i: (i, 0, 0)),
            pl.BlockSpec((NREL, D), lambda i: (0, 0)),
            pl.BlockSpec((K1B, 1), lambda i: (i, 0)),
            pl.BlockSpec((NREL, 1), lambda i: (0, 0)),
        ],
        out_shape=[
            jax.ShapeDtypeStruct((N, 2, DH), jnp.bfloat16),
            jax.ShapeDtypeStruct((NREL, D), f32),
            jax.ShapeDtypeStruct((N, 1), f32),
            jax.ShapeDtypeStruct((NREL, 1), f32),
        ],
    )(id_embed, ent, init_rel, W1, Wr1, att2, gtab, atab, ltab)


NSLOT = 5
GLEAD = 3
NBLKH = NBLK // 2


def _k2_body(xp2_hbm, sn_hbm, sr_hbm, src_hbm, combo_hbm,
             out_hbm, s_hbm,
             srcx_v, combo_v, dsti_s, sidx_s, ex_s, sn_v, sr_v,
             rowsb_s, rows_s, zlin_v, out_sh, acc_sh, semg, semr, sems):
    c = lax.axis_index("c")
    s = lax.axis_index("s")
    zero16 = jnp.zeros((16,), jnp.float32)
    rows_a_v = rows_s[0]

    @pl.loop(0, BLK)
    def _(r):
        for k in range(DH // 16):
            rows_a_v[r, pl.ds(k * 16, 16)] = zero16

    @pl.loop(0, 64)
    def _(i):
        zlin_v[pl.ds(i * 16, 16)] = zero16

    for k in range(7):
        pltpu.sync_copy(rows_a_v, out_sh.at[pl.ds(s * 625 + k * BLK, BLK)])
    pltpu.sync_copy(rows_a_v.at[pl.ds(0, 65)],
                    out_sh.at[pl.ds(s * 625 + 560, 65)])
    for k in range(8):
        pltpu.sync_copy(zlin_v.at[pl.ds(0, 640)],
                        acc_sh.at[pl.ds(s * 5120 + k * 640, 640)])

    pltpu.sync_copy(sn_hbm, sn_v)
    pltpu.sync_copy(sr_hbm, sr_v)

    plsc.subcore_barrier()

    def p_phase(b, ex_r, sidx_r, dsti_r):
        for k in range(BLK // 16):
            sl = pl.ds(k * 16, 16)
            src16 = lax.shift_right_logical(srcx_v[b, sl], 1)
            combo16 = combo_v[b, sl]
            dst16 = lax.shift_right_logical(combo16, 4)
            et16 = combo16 & 15
            sv = plsc.load_gather(sn_v, [src16])
            rv = plsc.load_gather(sr_v, [et16])
            lg = sv + rv
            lg = jnp.maximum(lg, lg * 0.2)
            ex_r[0, sl] = jnp.exp(lg)
            dsti_r[0, sl] = dst16
            mine = lax.shift_right_logical(et16, 3) == c
            sidx_r[0, sl] = jnp.where(
                mine, dst16 * 8 + (et16 & 7),
                jnp.full((16,), TRASH, jnp.int32))

    def c_phase(ex_r, rowsb_r, rows_r):
        @pl.loop(0, BLK)
        def _(j):
            a16 = plsc.load_gather(ex_r, [jnp.full((16,), 0, jnp.int32),
                                          jnp.full((16,), j, jnp.int32)])
            for g in range(DH // 32):
                pk = rowsb_r[j, pl.ds(g * 32, 32)]
                lo, hi = plsc.unpack(pk, format=plsc.PackFormat.INTERLEAVED,
                                     preferred_element_type=jnp.float32)
                rows_r[j, pl.ds(g * 32, 16)] = lo * a16
                rows_r[j, pl.ds(g * 32 + 16, 16)] = hi * a16

    def wait_row_scatter(slot):
        pltpu.make_async_copy(rows_s[slot], out_sh.at[dsti_s[0].at[0]],
                              semr).wait()

    def wait_small_scatter(slot):
        pltpu.make_async_copy(ex_s[slot].at[0], acc_sh.at[dsti_s[0].at[0]],
                              sems).wait()

    def issue_gather(b, slot):
        pltpu.async_copy(xp2_hbm.at[srcx_v.at[b]], rowsb_s[slot],
                         semg)

    for h in range(2):
        pltpu.sync_copy(src_hbm.at[s, pl.ds(h * NBLKH, NBLKH)], srcx_v)
        pltpu.sync_copy(combo_hbm.at[s, pl.ds(h * NBLKH, NBLKH)], combo_v)

        @pl.loop(0, NBLKH)
        def _(b):
            for k in range(BLK // 16):
                sl = pl.ds(k * 16, 16)
                srcx_v[b, sl] = srcx_v[b, sl] * 2 + c

        for t in range(GLEAD):
            issue_gather(t, t)

        @pl.loop(0, NBLKH // NSLOT)
        def _(i):
            b0 = i * NSLOT
            for t in range(NSLOT):
                b = b0 + t
                g = (t + GLEAD) % NSLOT
                bg = b + GLEAD

                @pl.when(b >= 2)
                def _():
                    wait_row_scatter(g)

                @pl.when(bg < NBLKH)
                def _():
                    issue_gather(bg, g)

                pltpu.make_async_copy(xp2_hbm.at[srcx_v.at[b]], rowsb_s[t],
                                      semg).wait()

                @pl.when(b >= NSLOT)
                def _():
                    wait_small_scatter(t)

                p_phase(b, ex_s[t], sidx_s[t], dsti_s[t])
                c_phase(ex_s[t], rowsb_s[t], rows_s[t])
                pltpu.async_copy(rows_s[t], out_sh.at[dsti_s[t].at[0]],
                                 semr, add=True)
                pltpu.async_copy(ex_s[t].at[0], acc_sh.at[sidx_s[t].at[0]],
                                 sems, add=True)

        wait_row_scatter((NBLKH - 2) % NSLOT)
        wait_row_scatter((NBLKH - 1) % NSLOT)
        for t in range(NSLOT):
            wait_small_scatter(t)

    plsc.subcore_barrier()

    for k in range(7):
        sl = pl.ds(s * 625 + k * BLK, BLK)
        pltpu.sync_copy(out_sh.at[sl], out_hbm.at[c, sl])
    slt = pl.ds(s * 625 + 560, 65)
    pltpu.sync_copy(out_sh.at[slt], out_hbm.at[c, slt])

    for k in range(8):
        sl = pl.ds(s * 5120 + k * 640, 640)
        pltpu.sync_copy(acc_sh.at[sl], s_hbm.at[c, sl])


def _k2(xp2, s_node, s_rel, src3, combo3):
    f32 = jnp.float32
    mesh = plsc.VectorSubcoreMesh(core_axis_name="c", subcore_axis_name="s")
    kern = pl.kernel(
        _k2_body,
        out_type=[
            jax.ShapeDtypeStruct((NC, NP, DH), f32),
            jax.ShapeDtypeStruct((NC, SP), f32),
        ],
        mesh=mesh,
        scratch_types=[
            pltpu.VMEM((NBLKH, BLK), jnp.int32),
            pltpu.VMEM((NBLKH, BLK), jnp.int32),
            [pltpu.VMEM((1, BLK), jnp.int32) for _ in range(NSLOT)],
            [pltpu.VMEM((1, BLK), jnp.int32) for _ in range(NSLOT)],
            [pltpu.VMEM((1, BLK), f32) for _ in range(NSLOT)],
            pltpu.VMEM((N,), f32),
            pltpu.VMEM((NREL,), f32),
            [pltpu.VMEM((BLK, DH), jnp.bfloat16) for _ in range(NSLOT)],
            [pltpu.VMEM((BLK, DH), f32) for _ in range(NSLOT)],
            pltpu.VMEM((1024,), f32),
            pltpu.VMEM_SHARED((NP, DH), f32),
            pltpu.VMEM_SHARED((SP,), f32),
            pltpu.SemaphoreType.DMA,
            pltpu.SemaphoreType.DMA,
            pltpu.SemaphoreType.DMA,
        ],
        compiler_params=_sc_compiler_params(),
    )
    return kern(xp2, s_node, s_rel, src3, combo3)


def _k3_body(ou_ref, s2_ref, rp_ref, rel_ref, x_ref, rel_emb_ref):
    f32 = jnp.float32
    rp = rp_ref[...]
    smat = jnp.concatenate([s2_ref[0], s2_ref[1]], axis=-1)
    srp = jnp.dot(smat, rp, preferred_element_type=f32)
    den = jnp.sum(smat, axis=1, keepdims=True) + 1e-16
    ou = jnp.concatenate([ou_ref[0], ou_ref[1]], axis=-1)
    x_ref[...] = jnp.tanh((ou + srp) / den)
    ohr = (rel_ref[...] == lax.broadcasted_iota(jnp.int32, (1, NREL), 1)
           ).astype(f32)
    rel_emb_ref[...] = jnp.dot(ohr, rp, preferred_element_type=f32)


def _k3(ou, S2, rp, rel2):
    f32 = jnp.float32
    return pl.pallas_call(
        _k3_body,
        out_shape=[
            jax.ShapeDtypeStruct((N, D), f32),
            jax.ShapeDtypeStruct((B, D), f32),
        ],
    )(ou, S2, rp, rel2)


def _k4_body(x_hbm, sub_hbm, out_hbm, idx_v, rows_v, sem):
    c = lax.axis_index("c")
    s = lax.axis_index("s")
    w = c * NS + s
    pltpu.sync_copy(sub_hbm.at[w], idx_v)
    pltpu.async_copy(x_hbm.at[idx_v], rows_v, sem).wait()
    pltpu.sync_copy(rows_v, out_hbm.at[pl.ds(w * (B // NW), B // NW)])


def _k4(x, sub3):
    f32 = jnp.float32
    bpw = B // NW
    mesh = plsc.VectorSubcoreMesh(core_axis_name="c", subcore_axis_name="s")
    kern = pl.kernel(
        _k4_body,
        out_type=jax.ShapeDtypeStruct((B, D), f32),
        mesh=mesh,
        scratch_types=[
            pltpu.VMEM((bpw,), jnp.int32),
            pltpu.VMEM((bpw, D), f32),
            pltpu.SemaphoreType.DMA,
        ],
        compiler_params=_sc_compiler_params(),
    )
    return kern(x, sub3)


def kernel(id_embed, gender_table, age_table, level_table, init_rel,
           W1, Wr1, att1, ent_feature, edge_index, edge_type, sub, rel):
    i32 = jnp.int32
    ent = ent_feature.astype(i32)
    perm = jnp.asarray(_PERM)
    att2 = att1[perm].reshape(D, 1)
    W1p = W1[:, perm]

    xph, rp, sn2, sr2 = _k1(id_embed, ent, init_rel, W1p, Wr1, att2,
                            gender_table, age_table, level_table)
    xp2 = xph.reshape(2 * N, DH)

    src3 = edge_index[0].astype(i32).reshape(NS, NBLK, BLK)
    combo3 = (edge_index[1].astype(i32) * NREL
              + edge_type.astype(i32)).reshape(NS, NBLK, BLK)
    s_node = sn2.reshape(N)
    s_rel = sr2.reshape(NREL)

    out_un, S_p = _k2(xp2, s_node, s_rel, src3, combo3)

    S2 = S_p[:, :N * 8].reshape(NC, N, 8)
    rel2 = rel.astype(i32).reshape(B, 1)

    x, rel_emb = _k3(out_un, S2, rp, rel2)

    sub3 = sub.astype(i32).reshape(NW, B // NW)
    sub_emb = _k4(x, sub3)

    return (sub_emb, rel_emb, x)


# speedup vs baseline: 1.1889x; 1.1889x over previous
import dataclasses

import jax
import jax.numpy as jnp
from jax import lax
from jax.experimental import pallas as pl
from jax.experimental.pallas import tpu as pltpu
from jax.experimental.pallas import tpu_sc as plsc

N = 10000
E = 320000
NP = 10000
D = 128
DH = D // 2
NREL = 16
B = 4096

NC, NS = 2, 16
NW = NC * NS
EPS = E // NS
BLK = 80
NBLK = EPS // BLK
SP = 81920
TRASH = 80000


def _sc_compiler_params():
    cp = pltpu.CompilerParams()
    fields = pltpu.CompilerParams.__dataclass_fields__
    if "needs_layout_passes" in fields:
        cp = dataclasses.replace(cp, needs_layout_passes=False)
    if "use_tc_tiling_on_sc" in fields:
        cp = dataclasses.replace(cp, use_tc_tiling_on_sc=False)
    return cp


K1B = 1000


def _k1_body(id_ref, ent_ref, ir_ref, w1_ref, wr_ref, att_ref,
             gt_ref, at_ref, lt_ref, xph_ref, rp_ref, sn_ref, sr_ref):
    f32 = jnp.float32
    w1 = w1_ref[...]
    gp = jnp.dot(gt_ref[...], w1[32:64, :], preferred_element_type=f32)
    ap = jnp.dot(at_ref[...], w1[64:96, :], preferred_element_type=f32)
    lp = jnp.dot(lt_ref[...], w1[96:128, :], preferred_element_type=f32)
    ent = ent_ref[...]
    ohg = (ent[:, 0:1] == lax.broadcasted_iota(jnp.int32, (1, 3), 1)
           ).astype(f32)
    oha = (ent[:, 1:2] == lax.broadcasted_iota(jnp.int32, (1, 9), 1)
           ).astype(f32)
    ohl = (ent[:, 2:3] == lax.broadcasted_iota(jnp.int32, (1, 11), 1)
           ).astype(f32)
    xp = (jnp.dot(id_ref[...], w1[0:32, :], preferred_element_type=f32)
          + jnp.dot(ohg, gp, preferred_element_type=f32)
          + jnp.dot(oha, ap, preferred_element_type=f32)
          + jnp.dot(ohl, lp, preferred_element_type=f32))
    att = att_ref[...]
    xph_ref[:, 0, :] = xp[:, :DH]
    xph_ref[:, 1, :] = xp[:, DH:]
    sn_ref[...] = jnp.dot(xp, att, preferred_element_type=f32)

    @pl.when(pl.program_id(0) == 0)
    def _():
        rp = jnp.dot(ir_ref[...], wr_ref[...], preferred_element_type=f32)
        rp_ref[...] = rp
        sr_ref[...] = jnp.dot(rp, att, preferred_element_type=f32)


def _k1(id_embed, ent, init_rel, W1, Wr1, att2, gtab, atab, ltab):
    f32 = jnp.float32
    g = N // K1B
    return pl.pallas_call(
        _k1_body,
        grid=(g,),
        in_specs=[
            pl.BlockSpec((K1B, 32), lambda i: (i, 0)),
            pl.BlockSpec((K1B, 3), lambda i: (i, 0)),
            pl.BlockSpec((NREL, D), lambda i: (0, 0)),
            pl.BlockSpec((D, D), lambda i: (0, 0)),
            pl.BlockSpec((D, D), lambda i: (0, 0)),
            pl.BlockSpec((D, 1), lambda i: (0, 0)),
            pl.BlockSpec((3, 32), lambda i: (0, 0)),
            pl.BlockSpec((9, 32), lambda i: (0, 0)),
            pl.BlockSpec((11, 32), lambda i: (0, 0)),
        ],
        out_specs=[
            pl.BlockSpec((K1B, 2, DH), lambda i: (i, 0, 0)),
            pl.BlockSpec((NREL, D), lambda i: (0, 0)),
            pl.BlockSpec((K1B, 1), lambda i: (i, 0)),
            pl.BlockSpec((NREL, 1), lambda i: (0, 0)),
        ],
        out_shape=[
            jax.ShapeDtypeStruct((N, 2, DH), f32),
            jax.ShapeDtypeStruct((NREL, D), f32),
            jax.ShapeDtypeStruct((N, 1), f32),
            jax.ShapeDtypeStruct((NREL, 1), f32),
        ],
    )(id_embed, ent, init_rel, W1, Wr1, att2, gtab, atab, ltab)


NSLOT = 5
GLEAD = 3


def _k2_body(xp2_hbm, sn_hbm, sr_hbm, src_hbm, combo_hbm,
             out_hbm, s_hbm,
             srcx_v, combo_v, dsti_s, sidx_s, ex_s, sn_v, sr_v,
             rows_s, zlin_v, out_sh, acc_sh, semg, semr, sems):
    c = lax.axis_index("c")
    s = lax.axis_index("s")
    zero16 = jnp.zeros((16,), jnp.float32)
    rows_a_v = rows_s[0]

    @pl.loop(0, BLK)
    def _(r):
        for k in range(DH // 16):
            rows_a_v[r, pl.ds(k * 16, 16)] = zero16

    @pl.loop(0, 64)
    def _(i):
        zlin_v[pl.ds(i * 16, 16)] = zero16

    for k in range(7):
        pltpu.sync_copy(rows_a_v, out_sh.at[pl.ds(s * 625 + k * BLK, BLK)])
    pltpu.sync_copy(rows_a_v.at[pl.ds(0, 65)],
                    out_sh.at[pl.ds(s * 625 + 560, 65)])
    for k in range(8):
        pltpu.sync_copy(zlin_v.at[pl.ds(0, 640)],
                        acc_sh.at[pl.ds(s * 5120 + k * 640, 640)])

    pltpu.sync_copy(src_hbm.at[s], srcx_v)
    pltpu.sync_copy(combo_hbm.at[s], combo_v)
    pltpu.sync_copy(sn_hbm, sn_v)
    pltpu.sync_copy(sr_hbm, sr_v)

    plsc.subcore_barrier()

    @pl.loop(0, NBLK)
    def _(b):
        for k in range(BLK // 16):
            sl = pl.ds(k * 16, 16)
            srcx_v[b, sl] = srcx_v[b, sl] * 2 + c

    def p_phase(b, ex_r, sidx_r, dsti_r):
        for k in range(BLK // 16):
            sl = pl.ds(k * 16, 16)
            src16 = lax.shift_right_logical(srcx_v[b, sl], 1)
            combo16 = combo_v[b, sl]
            dst16 = lax.shift_right_logical(combo16, 4)
            et16 = combo16 & 15
            sv = plsc.load_gather(sn_v, [src16])
            rv = plsc.load_gather(sr_v, [et16])
            lg = sv + rv
            lg = jnp.maximum(lg, lg * 0.2)
            ex_r[0, sl] = jnp.exp(lg)
            dsti_r[0, sl] = dst16
            mine = lax.shift_right_logical(et16, 3) == c
            sidx_r[0, sl] = jnp.where(
                mine, dst16 * 8 + (et16 & 7),
                jnp.full((16,), TRASH, jnp.int32))

    def c_phase(ex_r, rows_r):
        @pl.loop(0, BLK)
        def _(j):
            a16 = plsc.load_gather(ex_r, [jnp.full((16,), 0, jnp.int32),
                                          jnp.full((16,), j, jnp.int32)])
            for k in range(DH // 16):
                sl = pl.ds(k * 16, 16)
                rows_r[j, sl] = rows_r[j, sl] * a16

    def wait_row_scatter(slot):
        pltpu.make_async_copy(rows_s[slot], out_sh.at[dsti_s[0].at[0]],
                              semr).wait()

    def wait_small_scatter(slot):
        pltpu.make_async_copy(ex_s[slot].at[0], acc_sh.at[dsti_s[0].at[0]],
                              sems).wait()

    def issue_gather(b, slot):
        pltpu.async_copy(xp2_hbm.at[srcx_v.at[b]], rows_s[slot],
                         semg)

    for t in range(GLEAD):
        issue_gather(t, t)

    @pl.loop(0, NBLK // NSLOT)
    def _(i):
        b0 = i * NSLOT
        for t in range(NSLOT):
            b = b0 + t
            g = (t + GLEAD) % NSLOT
            bg = b + GLEAD

            @pl.when(b >= 2)
            def _():
                wait_row_scatter(g)

            @pl.when(bg < NBLK)
            def _():
                issue_gather(bg, g)

            pltpu.make_async_copy(xp2_hbm.at[srcx_v.at[b]], rows_s[t],
                                  semg).wait()

            @pl.when(b >= NSLOT)
            def _():
                wait_small_scatter(t)

            p_phase(b, ex_s[t], sidx_s[t], dsti_s[t])
            c_phase(ex_s[t], rows_s[t])
            pltpu.async_copy(rows_s[t], out_sh.at[dsti_s[t].at[0]], semr,
                             add=True)
            pltpu.async_copy(ex_s[t].at[0], acc_sh.at[sidx_s[t].at[0]],
                             sems, add=True)

    wait_row_scatter((NBLK - 2) % NSLOT)
    wait_row_scatter((NBLK - 1) % NSLOT)
    for t in range(NSLOT):
        wait_small_scatter(t)

    plsc.subcore_barrier()

    for k in range(7):
        sl = pl.ds(s * 625 + k * BLK, BLK)
        pltpu.sync_copy(out_sh.at[sl], out_hbm.at[c, sl])
    slt = pl.ds(s * 625 + 560, 65)
    pltpu.sync_copy(out_sh.at[slt], out_hbm.at[c, slt])

    for k in range(8):
        sl = pl.ds(s * 5120 + k * 640, 640)
        pltpu.sync_copy(acc_sh.at[sl], s_hbm.at[c, sl])


def _k2(xp2, s_node, s_rel, src3, combo3):
    f32 = jnp.float32
    mesh = plsc.VectorSubcoreMesh(core_axis_name="c", subcore_axis_name="s")
    kern = pl.kernel(
        _k2_body,
        out_type=[
            jax.ShapeDtypeStruct((NC, NP, DH), f32),
            jax.ShapeDtypeStruct((NC, SP), f32),
        ],
        mesh=mesh,
        scratch_types=[
            pltpu.VMEM((NBLK, BLK), jnp.int32),
            pltpu.VMEM((NBLK, BLK), jnp.int32),
            [pltpu.VMEM((1, BLK), jnp.int32) for _ in range(NSLOT)],
            [pltpu.VMEM((1, BLK), jnp.int32) for _ in range(NSLOT)],
            [pltpu.VMEM((1, BLK), f32) for _ in range(NSLOT)],
            pltpu.VMEM((N,), f32),
            pltpu.VMEM((NREL,), f32),
            [pltpu.VMEM((BLK, DH), f32) for _ in range(NSLOT)],
            pltpu.VMEM((1024,), f32),
            pltpu.VMEM_SHARED((NP, DH), f32),
            pltpu.VMEM_SHARED((SP,), f32),
            pltpu.SemaphoreType.DMA,
            pltpu.SemaphoreType.DMA,
            pltpu.SemaphoreType.DMA,
        ],
        compiler_params=_sc_compiler_params(),
    )
    return kern(xp2, s_node, s_rel, src3, combo3)


def _k3_body(ou_ref, s2_ref, rp_ref, rel_ref, x_ref, rel_emb_ref):
    f32 = jnp.float32
    rp = rp_ref[...]
    smat = jnp.concatenate([s2_ref[0], s2_ref[1]], axis=-1)
    srp = jnp.dot(smat, rp, preferred_element_type=f32)
    den = jnp.sum(smat, axis=1, keepdims=True) + 1e-16
    ou = jnp.concatenate([ou_ref[0], ou_ref[1]], axis=-1)
    x_ref[...] = jnp.tanh((ou + srp) / den)
    ohr = (rel_ref[...] == lax.broadcasted_iota(jnp.int32, (1, NREL), 1)
           ).astype(f32)
    rel_emb_ref[...] = jnp.dot(ohr, rp, preferred_element_type=f32)


def _k3(ou, S2, rp, rel2):
    f32 = jnp.float32
    return pl.pallas_call(
        _k3_body,
        out_shape=[
            jax.ShapeDtypeStruct((N, D), f32),
            jax.ShapeDtypeStruct((B, D), f32),
        ],
    )(ou, S2, rp, rel2)


def _k4_body(x_hbm, sub_hbm, out_hbm, idx_v, rows_v, sem):
    c = lax.axis_index("c")
    s = lax.axis_index("s")
    w = c * NS + s
    pltpu.sync_copy(sub_hbm.at[w], idx_v)
    pltpu.async_copy(x_hbm.at[idx_v], rows_v, sem).wait()
    pltpu.sync_copy(rows_v, out_hbm.at[pl.ds(w * (B // NW), B // NW)])


def _k4(x, sub3):
    f32 = jnp.float32
    bpw = B // NW
    mesh = plsc.VectorSubcoreMesh(core_axis_name="c", subcore_axis_name="s")
    kern = pl.kernel(
        _k4_body,
        out_type=jax.ShapeDtypeStruct((B, D), f32),
        mesh=mesh,
        scratch_types=[
            pltpu.VMEM((bpw,), jnp.int32),
            pltpu.VMEM((bpw, D), f32),
            pltpu.SemaphoreType.DMA,
        ],
        compiler_params=_sc_compiler_params(),
    )
    return kern(x, sub3)


def kernel(id_embed, gender_table, age_table, level_table, init_rel,
           W1, Wr1, att1, ent_feature, edge_index, edge_type, sub, rel):
    i32 = jnp.int32
    ent = ent_feature.astype(i32)
    att2 = att1.reshape(D, 1)

    xph, rp, sn2, sr2 = _k1(id_embed, ent, init_rel, W1, Wr1, att2,
                            gender_table, age_table, level_table)
    xp2 = xph.reshape(2 * N, DH)

    src3 = edge_index[0].astype(i32).reshape(NS, NBLK, BLK)
    combo3 = (edge_index[1].astype(i32) * NREL
              + edge_type.astype(i32)).reshape(NS, NBLK, BLK)
    s_node = sn2.reshape(N)
    s_rel = sr2.reshape(NREL)

    out_un, S_p = _k2(xp2, s_node, s_rel, src3, combo3)

    S2 = S_p[:, :N * 8].reshape(NC, N, 8)
    rel2 = rel.astype(i32).reshape(B, 1)

    x, rel_emb = _k3(out_un, S2, rp, rel2)

    sub3 = sub.astype(i32).reshape(NW, B // NW)
    sub_emb = _k4(x, sub3)

    return (sub_emb, rel_emb, x)
